# baseline (speedup 1.0000x reference)
import functools

import jax
import jax.numpy as jnp
from jax import lax
from jax.experimental import pallas as pl
from jax.experimental.pallas import tpu as pltpu
from jax.experimental.pallas import tpu_sc as plsc

_VOCAB = 100000
_DIM = 64
_BATCH = 4096
_HIST = 50

_NUM_WORKERS = 32
_ROWS_W = _BATCH // _NUM_WORKERS
_L = 16


def _transpose_block(gbuf, tbuf, rows_list):
    @plsc.parallel_loop(0, 8, unroll=2)
    def _t(st):
        base = st * 8
        for s in range(8):
            col = lax.broadcast(base + s, (_L,))
            for lb in range(8):
                v = plsc.load_gather(gbuf, [rows_list[lb], col])
                tbuf[st, pl.ds(s * 128 + lb * _L, _L)] = v


def _emb_body(idx_hbm, table_hbm, out_hbm,
              idx_v, gbuf_a, gbuf_b, tbuf_a, tbuf_b,
              gsems, osems):
    wid = lax.axis_index("s") * 2 + lax.axis_index("c")

    pltpu.sync_copy(idx_hbm.at[wid], idx_v)

    def gather(h, gbuf, gs):
        pltpu.async_copy(table_hbm.at[idx_v.at[h]], gbuf, gsems.at[gs])

    def wait_gather(gbuf, gs):
        pltpu.make_async_copy(
            table_hbm.at[idx_v.at[0]], gbuf, gsems.at[gs]
        ).wait()

    def put(h, tbuf, os):
        pltpu.async_copy(tbuf, out_hbm.at[h, :, wid], osems.at[os])

    def wait_put(tbuf, os):
        pltpu.make_async_copy(tbuf, out_hbm.at[0, :, wid], osems.at[os]).wait()

    iota = lax.iota(jnp.int32, _L)
    rows_list = [lb * _L + iota for lb in range(8)]

    gather(0, gbuf_a, 0)

    def body(i, carry):
        h0 = 2 * i
        gather(h0 + 1, gbuf_b, 1)
        wait_gather(gbuf_a, 0)

        @pl.when(i > 0)
        def _drain_a():
            wait_put(tbuf_a, 0)

        _transpose_block(gbuf_a, tbuf_a, rows_list)
        put(h0, tbuf_a, 0)

        @pl.when(h0 + 2 < _HIST)
        def _next_a():
            gather(h0 + 2, gbuf_a, 0)

        wait_gather(gbuf_b, 1)

        @pl.when(i > 0)
        def _drain_b():
            wait_put(tbuf_b, 1)

        _transpose_block(gbuf_b, tbuf_b, rows_list)
        put(h0 + 1, tbuf_b, 1)
        return carry

    lax.fori_loop(0, _HIST // 2, body, 0)

    wait_put(tbuf_a, 0)
    wait_put(tbuf_b, 1)


_emb_kernel = functools.partial(
    pl.kernel,
    out_type=jax.ShapeDtypeStruct((_HIST, 8, _NUM_WORKERS, 1024), jnp.float32),
    mesh=plsc.VectorSubcoreMesh(core_axis_name="c", subcore_axis_name="s"),
    scratch_types=[
        pltpu.VMEM((_HIST, _ROWS_W), jnp.int32),
        pltpu.VMEM((_ROWS_W, _DIM), jnp.float32),
        pltpu.VMEM((_ROWS_W, _DIM), jnp.float32),
        pltpu.VMEM((8, 1024), jnp.float32),
        pltpu.VMEM((8, 1024), jnp.float32),
        pltpu.SemaphoreType.DMA((2,)),
        pltpu.SemaphoreType.DMA((2,)),
    ],
    compiler_params=pltpu.CompilerParams(
        use_tc_tiling_on_sc=False, needs_layout_passes=False
    ),
)(_emb_body)


def kernel(indices, embedding):
    idx_t = indices.reshape(_NUM_WORKERS, _ROWS_W, _HIST).transpose(0, 2, 1)
    out5 = _emb_kernel(idx_t, embedding)
    out5 = out5.reshape(_HIST, 8, _NUM_WORKERS, 8, 128)
    return out5.transpose(2, 4, 0, 1, 3).reshape(_BATCH, _HIST, _DIM)


# speedup vs baseline: 1.1241x; 1.1241x over previous
import functools

import jax
import jax.numpy as jnp
from jax import lax
from jax.experimental import pallas as pl
from jax.experimental.pallas import tpu as pltpu
from jax.experimental.pallas import tpu_sc as plsc

_VOCAB = 100000
_DIM = 64
_BATCH = 4096
_HIST = 50

_NUM_WORKERS = 32
_ROWS_W = _BATCH // _NUM_WORKERS
_L = 16


def _transpose_block(gbuf, tbuf, rows_list):
    @plsc.parallel_loop(0, _DIM, unroll=16)
    def _t(k):
        st = lax.shift_right_logical(k, 3)
        s = lax.bitwise_and(k, 7)
        off = s * 128
        col = lax.broadcast(k, (_L,))
        for lb in range(8):
            v = plsc.load_gather(gbuf, [rows_list[lb], col])
            tbuf[st, pl.ds(off + lb * _L, _L)] = v


def _emb_body(idx_hbm, table_hbm, out_hbm,
              idx_v, gbuf_a, gbuf_b, tbuf_a, tbuf_b,
              gsems, osems):
    wid = lax.axis_index("s") * 2 + lax.axis_index("c")

    pltpu.sync_copy(idx_hbm.at[wid], idx_v)

    def gather(h, gbuf, gs):
        pltpu.async_copy(table_hbm.at[idx_v.at[h]], gbuf, gsems.at[gs])

    def wait_gather(gbuf, gs):
        pltpu.make_async_copy(
            table_hbm.at[idx_v.at[0]], gbuf, gsems.at[gs]
        ).wait()

    def put(h, tbuf, os):
        pltpu.async_copy(tbuf, out_hbm.at[h, :, wid], osems.at[os])

    def wait_put(tbuf, os):
        pltpu.make_async_copy(tbuf, out_hbm.at[0, :, wid], osems.at[os]).wait()

    iota = lax.iota(jnp.int32, _L)
    rows_list = [lb * _L + iota for lb in range(8)]

    gather(0, gbuf_a, 0)

    def body(i, carry):
        h0 = 2 * i
        gather(h0 + 1, gbuf_b, 1)
        wait_gather(gbuf_a, 0)

        @pl.when(i > 0)
        def _drain_a():
            wait_put(tbuf_a, 0)

        _transpose_block(gbuf_a, tbuf_a, rows_list)
        put(h0, tbuf_a, 0)

        @pl.when(h0 + 2 < _HIST)
        def _next_a():
            gather(h0 + 2, gbuf_a, 0)

        wait_gather(gbuf_b, 1)

        @pl.when(i > 0)
        def _drain_b():
            wait_put(tbuf_b, 1)

        _transpose_block(gbuf_b, tbuf_b, rows_list)
        put(h0 + 1, tbuf_b, 1)
        return carry

    lax.fori_loop(0, _HIST // 2, body, 0)

    wait_put(tbuf_a, 0)
    wait_put(tbuf_b, 1)


_emb_kernel = functools.partial(
    pl.kernel,
    out_type=jax.ShapeDtypeStruct((_HIST, 8, _NUM_WORKERS, 1024), jnp.float32),
    mesh=plsc.VectorSubcoreMesh(core_axis_name="c", subcore_axis_name="s"),
    scratch_types=[
        pltpu.VMEM((_HIST, _ROWS_W), jnp.int32),
        pltpu.VMEM((_ROWS_W, _DIM), jnp.float32),
        pltpu.VMEM((_ROWS_W, _DIM), jnp.float32),
        pltpu.VMEM((8, 1024), jnp.float32),
        pltpu.VMEM((8, 1024), jnp.float32),
        pltpu.SemaphoreType.DMA((2,)),
        pltpu.SemaphoreType.DMA((2,)),
    ],
    compiler_params=pltpu.CompilerParams(
        use_tc_tiling_on_sc=False, needs_layout_passes=False
    ),
)(_emb_body)


def kernel(indices, embedding):
    idx_t = indices.reshape(_NUM_WORKERS, _ROWS_W, _HIST).transpose(0, 2, 1)
    out5 = _emb_kernel(idx_t, embedding)
    out5 = out5.reshape(_HIST, 8, _NUM_WORKERS, 8, 128)
    return out5.transpose(2, 4, 0, 1, 3).reshape(_BATCH, _HIST, _DIM)


# speedup vs baseline: 2.1804x; 1.9397x over previous
import functools

import jax
import jax.numpy as jnp
from jax import lax
from jax.experimental import pallas as pl
from jax.experimental.pallas import tpu as pltpu
from jax.experimental.pallas import tpu_sc as plsc

_VOCAB = 100000
_DIM = 64
_BATCH = 4096
_HIST = 50

_NUM_WORKERS = 32
_ROWS_W = _BATCH // _NUM_WORKERS
_L = 16


def _transpose_block(gbuf, tbuf, st_idx, s_idx):
    @plsc.parallel_loop(0, _ROWS_W, unroll=8)
    def _t(l):
        lv = lax.broadcast(l, (_L,))
        for k in range(4):
            v = gbuf[l, pl.ds(k * _L, _L)]
            plsc.store_scatter(tbuf, [st_idx[k], s_idx[k], lv], v)


def _emb_body(idx_hbm, table_hbm, out_hbm,
              idx_v, gbuf_a, gbuf_b, tbuf_a, tbuf_b,
              gsems, osems):
    wid = lax.axis_index("s") * 2 + lax.axis_index("c")

    pltpu.sync_copy(idx_hbm.at[wid], idx_v)

    def gather(h, gbuf, gs):
        pltpu.async_copy(table_hbm.at[idx_v.at[h]], gbuf, gsems.at[gs])

    def wait_gather(gbuf, gs):
        pltpu.make_async_copy(
            table_hbm.at[idx_v.at[0]], gbuf, gsems.at[gs]
        ).wait()

    def put(h, tbuf, os):
        pltpu.async_copy(
            tbuf.at[:, :, pl.ds(0, 128)], out_hbm.at[h, :, wid], osems.at[os]
        )

    def wait_put(tbuf, os):
        pltpu.make_async_copy(
            tbuf.at[:, :, pl.ds(0, 128)], out_hbm.at[0, :, wid], osems.at[os]
        ).wait()

    iota = lax.iota(jnp.int32, _L)
    st_idx = [lax.shift_right_logical(k * _L + iota, 3) for k in range(4)]
    s_idx = [lax.bitwise_and(k * _L + iota, 7) for k in range(4)]

    gather(0, gbuf_a, 0)

    def body(i, carry):
        h0 = 2 * i
        gather(h0 + 1, gbuf_b, 1)
        wait_gather(gbuf_a, 0)

        @pl.when(i > 0)
        def _drain_a():
            wait_put(tbuf_a, 0)

        _transpose_block(gbuf_a, tbuf_a, st_idx, s_idx)
        put(h0, tbuf_a, 0)

        @pl.when(h0 + 2 < _HIST)
        def _next_a():
            gather(h0 + 2, gbuf_a, 0)

        wait_gather(gbuf_b, 1)

        @pl.when(i > 0)
        def _drain_b():
            wait_put(tbuf_b, 1)

        _transpose_block(gbuf_b, tbuf_b, st_idx, s_idx)
        put(h0 + 1, tbuf_b, 1)
        return carry

    lax.fori_loop(0, _HIST // 2, body, 0)

    wait_put(tbuf_a, 0)
    wait_put(tbuf_b, 1)


_emb_kernel = functools.partial(
    pl.kernel,
    out_type=jax.ShapeDtypeStruct((_HIST, 8, _NUM_WORKERS, 8, 128), jnp.float32),
    mesh=plsc.VectorSubcoreMesh(core_axis_name="c", subcore_axis_name="s"),
    scratch_types=[
        pltpu.VMEM((_HIST, _ROWS_W), jnp.int32),
        pltpu.VMEM((_ROWS_W, _DIM), jnp.float32),
        pltpu.VMEM((_ROWS_W, _DIM), jnp.float32),
        pltpu.VMEM((8, 8, 137), jnp.float32),
        pltpu.VMEM((8, 8, 137), jnp.float32),
        pltpu.SemaphoreType.DMA((2,)),
        pltpu.SemaphoreType.DMA((2,)),
    ],
    compiler_params=pltpu.CompilerParams(
        use_tc_tiling_on_sc=False, needs_layout_passes=False
    ),
)(_emb_body)


def kernel(indices, embedding):
    idx_t = indices.reshape(_NUM_WORKERS, _ROWS_W, _HIST).transpose(0, 2, 1)
    out5 = _emb_kernel(idx_t, embedding)
    return out5.transpose(2, 4, 0, 1, 3).reshape(_BATCH, _HIST, _DIM)
